# baseline (speedup 1.0000x reference)
import functools

import jax
import jax.numpy as jnp
from jax import lax
from jax.experimental import pallas as pl
from jax.experimental.pallas import tpu as pltpu
from jax.experimental.pallas import tpu_sc as plsc

EMBED_DIM = 128
CHUNK = 128
NBUF = 5


@functools.partial(jax.jit, static_argnames=("num_workers", "n_chunks"))
def _sc_embed_lookup(idx, table, *, num_workers, n_chunks):
    mesh = plsc.VectorSubcoreMesh(core_axis_name="c", subcore_axis_name="s")
    num_cores = mesh.num_cores
    total_rows = num_workers * n_chunks * CHUNK
    rows_per_worker = n_chunks * CHUNK

    n_iters = n_chunks // NBUF

    @functools.partial(
        pl.kernel,
        out_type=jax.ShapeDtypeStruct((total_rows, EMBED_DIM), jnp.float32),
        mesh=mesh,
        scratch_types=[
            pltpu.VMEM((n_chunks, CHUNK), jnp.int32),
            *[pltpu.VMEM((CHUNK, EMBED_DIM), jnp.float32) for _ in range(NBUF)],
            *[pltpu.SemaphoreType.DMA for _ in range(2 * NBUF)],
        ],
    )
    def body(idx_hbm, tab_hbm, out_hbm, idx_v, *scratch):
        bufs = scratch[:NBUF]
        sin = scratch[NBUF : 2 * NBUF]
        sout = scratch[2 * NBUF : 3 * NBUF]
        wid = lax.axis_index("s") * num_cores + lax.axis_index("c")
        base = wid * rows_per_worker
        pltpu.sync_copy(idx_hbm.at[wid], idx_v)
        for b in range(NBUF):
            pltpu.async_copy(tab_hbm.at[idx_v.at[b]], bufs[b], sin[b])

        def step(t, carry):
            j0 = t * NBUF
            for b in range(NBUF):
                pltpu.make_async_copy(
                    tab_hbm.at[idx_v.at[j0 + b]], bufs[b], sin[b]
                ).wait()
                pltpu.async_copy(
                    bufs[b],
                    out_hbm.at[pl.ds(base + (j0 + b) * CHUNK, CHUNK)],
                    sout[b],
                )

            @pl.when(t + 1 < n_iters)
            def _():
                for b in range(NBUF):
                    pltpu.make_async_copy(
                        bufs[b],
                        out_hbm.at[pl.ds(base + (j0 + b) * CHUNK, CHUNK)],
                        sout[b],
                    ).wait()
                    pltpu.async_copy(
                        tab_hbm.at[idx_v.at[j0 + NBUF + b]], bufs[b], sin[b]
                    )

            return carry

        lax.fori_loop(0, n_iters, step, 0)
        for b in range(NBUF):
            j = n_chunks - NBUF + b
            pltpu.make_async_copy(
                bufs[b], out_hbm.at[pl.ds(base + j * CHUNK, CHUNK)], sout[b]
            ).wait()

    return body(idx, table)


def kernel(inputs, embed_weight):
    if inputs.shape[-1] == 1:
        inputs = jnp.squeeze(inputs, axis=-1)
    lead_shape = inputs.shape
    flat = inputs.reshape(-1).astype(jnp.int32)
    n = flat.shape[0]
    num_workers = 32
    assert n % (num_workers * CHUNK) == 0
    n_chunks = n // (num_workers * CHUNK)
    assert n_chunks % NBUF == 0
    idx = flat.reshape(num_workers, n_chunks, CHUNK)
    out = _sc_embed_lookup(
        idx, embed_weight, num_workers=num_workers, n_chunks=n_chunks
    )
    return out.reshape(*lead_shape, EMBED_DIM)


# speedup vs baseline: 2.8848x; 2.8848x over previous
import functools

import jax
import jax.numpy as jnp
from jax import lax
from jax.experimental import pallas as pl
from jax.experimental.pallas import tpu as pltpu
from jax.experimental.pallas import tpu_sc as plsc

EMBED_DIM = 128
CHUNK = 128
NBUF = 2


@functools.partial(jax.jit, static_argnames=("num_workers", "n_chunks"))
def _sc_embed_lookup(idx, table, *, num_workers, n_chunks):
    mesh = plsc.VectorSubcoreMesh(core_axis_name="c", subcore_axis_name="s")
    num_cores = mesh.num_cores
    total_rows = num_workers * n_chunks * CHUNK
    rows_per_worker = n_chunks * CHUNK

    n_iters = n_chunks // NBUF

    @functools.partial(
        pl.kernel,
        out_type=jax.ShapeDtypeStruct((total_rows, EMBED_DIM), jnp.float32),
        mesh=mesh,
        scratch_types=[
            pltpu.VMEM((n_chunks, CHUNK), jnp.int32),
            *[pltpu.VMEM((CHUNK, EMBED_DIM), jnp.float32) for _ in range(NBUF)],
            *[pltpu.SemaphoreType.DMA for _ in range(2 * NBUF)],
        ],
    )
    def body(idx_hbm, tab_hbm, out_hbm, idx_v, *scratch):
        bufs = scratch[:NBUF]
        sin = scratch[NBUF : 2 * NBUF]
        sout = scratch[2 * NBUF : 3 * NBUF]
        wid = lax.axis_index("s") * num_cores + lax.axis_index("c")
        base = wid * rows_per_worker
        pltpu.sync_copy(idx_hbm.at[wid], idx_v)
        for b in range(NBUF):
            pltpu.async_copy(tab_hbm.at[idx_v.at[b]], bufs[b], sin[b])

        def step(t, carry):
            j0 = t * NBUF
            for b in range(NBUF):
                pltpu.make_async_copy(
                    tab_hbm.at[idx_v.at[j0 + b]], bufs[b], sin[b]
                ).wait()
                pltpu.async_copy(
                    bufs[b],
                    out_hbm.at[pl.ds(base + (j0 + b) * CHUNK, CHUNK)],
                    sout[b],
                )

            @pl.when(t + 1 < n_iters)
            def _():
                for b in range(NBUF):
                    pltpu.make_async_copy(
                        bufs[b],
                        out_hbm.at[pl.ds(base + (j0 + b) * CHUNK, CHUNK)],
                        sout[b],
                    ).wait()
                    pltpu.async_copy(
                        tab_hbm.at[idx_v.at[j0 + NBUF + b]], bufs[b], sin[b]
                    )

            return carry

        lax.fori_loop(0, n_iters, step, 0)
        for b in range(NBUF):
            j = n_chunks - NBUF + b
            pltpu.make_async_copy(
                bufs[b], out_hbm.at[pl.ds(base + j * CHUNK, CHUNK)], sout[b]
            ).wait()

    return body(idx, table)


def kernel(inputs, embed_weight):
    if inputs.shape[-1] == 1:
        inputs = jnp.squeeze(inputs, axis=-1)
    lead_shape = inputs.shape
    flat = inputs.T.reshape(-1).astype(jnp.int32)
    n = flat.shape[0]
    num_workers = 32
    assert n % (num_workers * CHUNK) == 0
    n_chunks = n // (num_workers * CHUNK)
    assert n_chunks % NBUF == 0
    idx = flat.reshape(num_workers, n_chunks, CHUNK)
    out = _sc_embed_lookup(
        idx, embed_weight, num_workers=num_workers, n_chunks=n_chunks
    )
    out = out.reshape(lead_shape[1], lead_shape[0], EMBED_DIM)
    return jnp.transpose(out, (1, 0, 2))


# speedup vs baseline: 3.1162x; 1.0802x over previous
import functools

import jax
import jax.numpy as jnp
from jax import lax
from jax.experimental import pallas as pl
from jax.experimental.pallas import tpu as pltpu
from jax.experimental.pallas import tpu_sc as plsc

EMBED_DIM = 128
CHUNK = 64
NBUF = 10


@functools.partial(jax.jit, static_argnames=("num_workers", "n_chunks"))
def _sc_embed_lookup(idx, table, *, num_workers, n_chunks):
    mesh = plsc.VectorSubcoreMesh(core_axis_name="c", subcore_axis_name="s")
    num_cores = mesh.num_cores
    total_rows = num_workers * n_chunks * CHUNK
    rows_per_worker = n_chunks * CHUNK

    n_iters = n_chunks // NBUF

    @functools.partial(
        pl.kernel,
        out_type=jax.ShapeDtypeStruct((total_rows, EMBED_DIM), jnp.float32),
        mesh=mesh,
        scratch_types=[
            pltpu.VMEM((n_chunks, CHUNK), jnp.int32),
            *[pltpu.VMEM((CHUNK, EMBED_DIM), jnp.float32) for _ in range(NBUF)],
            *[pltpu.SemaphoreType.DMA for _ in range(2 * NBUF)],
        ],
    )
    def body(idx_hbm, tab_hbm, out_hbm, idx_v, *scratch):
        bufs = scratch[:NBUF]
        sin = scratch[NBUF : 2 * NBUF]
        sout = scratch[2 * NBUF : 3 * NBUF]
        wid = lax.axis_index("s") * num_cores + lax.axis_index("c")
        base = wid * rows_per_worker
        pltpu.sync_copy(idx_hbm.at[wid], idx_v)
        for b in range(NBUF):
            pltpu.async_copy(tab_hbm.at[idx_v.at[b]], bufs[b], sin[b])

        def step(t, carry):
            j0 = t * NBUF
            for b in range(NBUF):
                pltpu.make_async_copy(
                    tab_hbm.at[idx_v.at[j0 + b]], bufs[b], sin[b]
                ).wait()
                pltpu.async_copy(
                    bufs[b],
                    out_hbm.at[pl.ds(base + (j0 + b) * CHUNK, CHUNK)],
                    sout[b],
                )

            @pl.when(t + 1 < n_iters)
            def _():
                for b in range(NBUF):
                    pltpu.make_async_copy(
                        bufs[b],
                        out_hbm.at[pl.ds(base + (j0 + b) * CHUNK, CHUNK)],
                        sout[b],
                    ).wait()
                    pltpu.async_copy(
                        tab_hbm.at[idx_v.at[j0 + NBUF + b]], bufs[b], sin[b]
                    )

            return carry

        lax.fori_loop(0, n_iters, step, 0)
        for b in range(NBUF):
            j = n_chunks - NBUF + b
            pltpu.make_async_copy(
                bufs[b], out_hbm.at[pl.ds(base + j * CHUNK, CHUNK)], sout[b]
            ).wait()

    return body(idx, table)


def kernel(inputs, embed_weight):
    if inputs.shape[-1] == 1:
        inputs = jnp.squeeze(inputs, axis=-1)
    lead_shape = inputs.shape
    flat = inputs.T.reshape(-1).astype(jnp.int32)
    n = flat.shape[0]
    num_workers = 32
    assert n % (num_workers * CHUNK) == 0
    n_chunks = n // (num_workers * CHUNK)
    assert n_chunks % NBUF == 0
    idx = flat.reshape(num_workers, n_chunks, CHUNK)
    out = _sc_embed_lookup(
        idx, embed_weight, num_workers=num_workers, n_chunks=n_chunks
    )
    out = out.reshape(lead_shape[1], lead_shape[0], EMBED_DIM)
    return jnp.transpose(out, (1, 0, 2))
